# Initial kernel scaffold; baseline (speedup 1.0000x reference)
import jax
import jax.numpy as jnp
from jax.experimental import pallas as pl


def kernel(interaction_pairs, table, W, b):
    raise NotImplementedError("write your pallas kernel here")


# speedup vs baseline: 1.4853x; 1.4853x over previous
import functools

import jax
import jax.numpy as jnp
from jax import lax
from jax.experimental import pallas as pl
from jax.experimental.pallas import tpu as pltpu
from jax.experimental.pallas import tpu_sc as plsc

B, F, V, D = 16384, 26, 1000000, 32
NC, NS = 2, 16
NW = NC * NS
RPW = B // NW
CH = 64
NCH = RPW // CH
CHF = CH * F
L = 16


def _fm_body(idx_hbm, table_hbm, w_hbm, out_hbm, idxbuf, ebuf, w_v, out_v, sem):
    cid = lax.axis_index("c")
    sid = lax.axis_index("s")
    wid = sid * NC + cid
    base = wid * RPW

    pltpu.sync_copy(w_hbm, w_v)
    lane = lax.iota(jnp.int32, L)
    w_lo = w_v[pl.ds(0, L)]
    w_hi = w_v[pl.ds(L, L)]
    w_scal = [w_lo[j] for j in range(L)] + [w_hi[j] for j in range(F - L)]
    b_scal = w_hi[15]

    def chunk_body(ch, _):
        off = pl.multiple_of((wid * NCH + ch) * CHF, CHF)
        pltpu.sync_copy(idx_hbm.at[pl.ds(off, CHF)], idxbuf)
        pltpu.async_copy(table_hbm.at[idxbuf], ebuf, sem).wait()

        def group_body(g, _):
            r0 = g * L
            rvec = (r0 + lane) * F
            rvecs = [rvec + f for f in range(F)]

            lin = jnp.zeros((L,), jnp.float32)
            for f in range(F):
                iv = plsc.load_gather(idxbuf, [rvecs[f]])
                lin = lin + iv.astype(jnp.float32) * w_scal[f]

            def d_body(d, carry):
                qacc, tacc = carry
                cvec = jnp.full((L,), d, jnp.int32)
                s = jnp.zeros((L,), jnp.float32)
                for f in range(F):
                    v = plsc.load_gather(ebuf, [rvecs[f], cvec])
                    s = s + v
                    qacc = qacc + v * v
                return qacc, tacc + s * s

            qacc, tacc = lax.fori_loop(
                0, D, d_body,
                (jnp.zeros((L,), jnp.float32), jnp.zeros((L,), jnp.float32)))

            res = lin + 0.5 * (tacc - qacc) + b_scal
            out_v[pl.ds(ch * CH + r0, L)] = res
            return 0

        lax.fori_loop(0, CH // L, group_body, 0)
        return 0

    lax.fori_loop(0, NCH, chunk_body, 0)
    pltpu.sync_copy(out_v, out_hbm.at[pl.ds(base, RPW)])


@functools.partial(jax.jit, static_argnames=())
def _fm_sc(idx_flat, table, wfull):
    run = pl.kernel(
        _fm_body,
        out_type=jax.ShapeDtypeStruct((B,), jnp.float32),
        mesh=plsc.VectorSubcoreMesh(
            core_axis_name="c", subcore_axis_name="s",
            num_cores=NC, num_subcores=NS),
        scratch_types=[
            pltpu.VMEM((CHF,), jnp.int32),
            pltpu.VMEM((CHF, D), jnp.float32),
            pltpu.VMEM((2 * L,), jnp.float32),
            pltpu.VMEM((RPW,), jnp.float32),
            pltpu.SemaphoreType.DMA,
        ],
        compiler_params=pltpu.CompilerParams(
            needs_layout_passes=False, use_tc_tiling_on_sc=False),
    )
    return run(idx_flat, table, wfull)


def kernel(interaction_pairs, table, W, b):
    idx = interaction_pairs.astype(jnp.int32)
    idx_flat = idx.reshape(B * F)
    wfull = jnp.concatenate(
        [W[:, 0].astype(jnp.float32),
         jnp.zeros((2 * L - F - 1,), jnp.float32),
         b.astype(jnp.float32)])
    return _fm_sc(idx_flat, table, wfull)
